# baseline (speedup 1.0000x reference)
import functools

import jax
import jax.numpy as jnp
from jax import lax
from jax.experimental import pallas as pl
from jax.experimental.pallas import tpu as pltpu
from jax.experimental.pallas import tpu_sc as plsc

_NC = 2
_NS = 16
_L = 16


def _build(B, F, V, D):
    NW = _NC * _NS
    BPW = B // NW
    C = 64
    NCH = BPW // C
    IPC = C * F
    NSUB = IPC // 128
    assert B % NW == 0 and BPW % C == 0 and IPC % 128 == 0

    mesh = plsc.VectorSubcoreMesh(core_axis_name="c", subcore_axis_name="s")

    @functools.partial(
        pl.kernel,
        mesh=mesh,
        compiler_params=pltpu.CompilerParams(
            needs_layout_passes=False, use_tc_tiling_on_sc=False),
        out_type=jax.ShapeDtypeStruct((B,), jnp.float32),
        scratch_types=[
            pltpu.VMEM((IPC,), jnp.int32),
            pltpu.VMEM((NSUB, 128), jnp.int32),
            pltpu.VMEM((IPC, _L), jnp.float32),
            pltpu.VMEM((IPC + _L,), jnp.float32),
            pltpu.VMEM((C,), jnp.float32),
            pltpu.VMEM((_L,), jnp.float32),
            pltpu.SemaphoreType.DMA,
        ],
    )
    def fm(x_hbm, emb_hbm, lin_hbm, lb_hbm, out_hbm,
           x_v, idx_v, rows_v, lin_v, out_v, lb_v, sem):
        cid = lax.axis_index("c")
        sid = lax.axis_index("s")
        wid = sid * _NC + cid
        base = wid * BPW

        pltpu.sync_copy(lb_hbm, lb_v.at[pl.ds(0, 1)])
        lb = lb_v[pl.ds(0, _L)][0]

        lane = lax.iota(jnp.int32, _L)
        m_tail = jnp.where(lane < (F - _L), 1.0, 0.0).astype(jnp.float32)
        m_last = lane == (_L - 1)
        zeros_i = jnp.zeros((_L,), jnp.int32)

        def chunk_body(ci, carry):
            cbase = base + ci * C
            pltpu.sync_copy(x_hbm.at[pl.ds(cbase * F, IPC)], x_v)

            def idx_body(r, carry2):
                for j in range(128 // _L):
                    p0 = r * 128 + j * _L
                    pos = p0 + lane
                    fld = lax.rem(pos, F)
                    idx_v[r, pl.ds(j * _L, _L)] = x_v[pl.ds(p0, _L)] + fld * V
                return carry2
            lax.fori_loop(0, NSUB, idx_body, 0)

            copies = []
            for j in range(NSUB):
                copies.append(pltpu.async_copy(
                    emb_hbm.at[idx_v.at[j]],
                    rows_v.at[pl.ds(j * 128, 128)], sem))
                copies.append(pltpu.async_copy(
                    lin_hbm.at[idx_v.at[j]],
                    lin_v.at[pl.ds(j * 128, 128)], sem))
            for cp in copies:
                cp.wait()

            def row_body(b, carry2):
                rbase = b * F
                acc = rows_v[rbase, :]
                acc2 = acc * acc
                for f in range(1, F):
                    r = rows_v[rbase + f, :]
                    acc = acc + r
                    acc2 = acc2 + r * r
                lv = lin_v[pl.ds(rbase, _L)] + lin_v[pl.ds(rbase + _L, _L)] * m_tail
                cs = plsc.cumsum(acc)
                cq = plsc.cumsum(acc2)
                cl = plsc.cumsum(lv)
                logit = cl + lb + 0.5 * (cs * cs - cq)
                sig = 1.0 / (1.0 + jnp.exp(-logit))
                plsc.store_scatter(out_v, [zeros_i + b], sig, mask=m_last)
                return carry2
            lax.fori_loop(0, C, row_body, 0)

            pltpu.sync_copy(out_v, out_hbm.at[pl.ds(cbase, C)])
            return carry
        lax.fori_loop(0, NCH, chunk_body, 0)

    return fm


def _build_reshape(FV, D):
    NW = _NC * _NS
    RW = FV // NW
    CR = 625
    NCH = RW // CR
    NP = NCH // 2
    UN = 5
    assert FV % NW == 0 and RW % CR == 0 and NCH % 2 == 0 and CR % UN == 0

    mesh = plsc.VectorSubcoreMesh(core_axis_name="c", subcore_axis_name="s")

    @functools.partial(
        pl.kernel,
        mesh=mesh,
        compiler_params=pltpu.CompilerParams(
            needs_layout_passes=False, use_tc_tiling_on_sc=False),
        out_type=jax.ShapeDtypeStruct((FV, D), jnp.float32),
        scratch_types=[
            pltpu.VMEM((2, CR * D), jnp.float32),
            pltpu.VMEM((2, CR, D), jnp.float32),
            pltpu.SemaphoreType.DMA,
            pltpu.SemaphoreType.DMA,
            pltpu.SemaphoreType.DMA,
            pltpu.SemaphoreType.DMA,
        ],
    )
    def relayout(src_hbm, dst_hbm, v1, v2, si0, si1, so0, so1):
        cid = lax.axis_index("c")
        sid = lax.axis_index("s")
        wid = sid * _NC + cid
        base = wid * RW
        sin = (si0, si1)
        sout = (so0, so1)

        def fire_in(ci, p):
            pltpu.async_copy(
                src_hbm.at[pl.ds((base + ci * CR) * D, CR * D)],
                v1.at[p], sin[p])

        def wait_in(ci, p):
            pltpu.make_async_copy(
                src_hbm.at[pl.ds((base + ci * CR) * D, CR * D)],
                v1.at[p], sin[p]).wait()

        def fire_out(ci, p):
            pltpu.async_copy(
                v2.at[p], dst_hbm.at[pl.ds(base + ci * CR, CR)], sout[p])

        def wait_out(ci, p):
            pltpu.make_async_copy(
                v2.at[p], dst_hbm.at[pl.ds(base + ci * CR, CR)], sout[p]).wait()

        def vcopy(p):
            def step(r, carry):
                r0 = r * UN
                for k in range(UN):
                    v2[p, r0 + k, :] = v1[p, pl.ds((r0 + k) * D, D)]
                return carry
            lax.fori_loop(0, CR // UN, step, 0)

        fire_in(0, 0)

        def pair_body(g, carry):
            c0 = 2 * g

            fire_in(c0 + 1, 1)
            wait_in(c0, 0)

            @pl.when(g > 0)
            def _():
                wait_out(c0 - 2, 0)
            vcopy(0)
            fire_out(c0, 0)

            @pl.when(g < NP - 1)
            def _():
                fire_in(c0 + 2, 0)
            wait_in(c0 + 1, 1)

            @pl.when(g > 0)
            def _():
                wait_out(c0 - 1, 1)
            vcopy(1)
            fire_out(c0 + 1, 1)
            return carry
        lax.fori_loop(0, NP, pair_body, 0)

        wait_out(NCH - 2, 0)
        wait_out(NCH - 1, 1)

    return relayout


def kernel(x, emb_table, lin_w, lin_b):
    B, F = x.shape
    D = emb_table.shape[1]
    V = emb_table.shape[0] // F
    emb_2d = _build_reshape(F * V, D)(emb_table.reshape(-1))
    fm = _build(B, F, V, D)
    out = fm(x.reshape(B * F).astype(jnp.int32), emb_2d, lin_w, lin_b)
    return out.reshape(B, 1)


# speedup vs baseline: 1.2550x; 1.2550x over previous
import functools

import jax
import jax.numpy as jnp
from jax import lax
from jax.experimental import pallas as pl
from jax.experimental.pallas import tpu as pltpu
from jax.experimental.pallas import tpu_sc as plsc

_NC = 2
_NS = 16
_L = 16


def _build(B, F, V, D):
    NW = _NC * _NS
    BPW = B // NW
    C = 64
    NCH = BPW // C
    IPC = C * F
    NSUB = IPC // 128
    assert B % NW == 0 and BPW % C == 0 and IPC % 128 == 0

    mesh = plsc.VectorSubcoreMesh(core_axis_name="c", subcore_axis_name="s")

    @functools.partial(
        pl.kernel,
        mesh=mesh,
        compiler_params=pltpu.CompilerParams(
            needs_layout_passes=False, use_tc_tiling_on_sc=False),
        out_type=jax.ShapeDtypeStruct((B,), jnp.float32),
        scratch_types=[
            pltpu.VMEM((IPC,), jnp.int32),
            pltpu.VMEM((NSUB, 128), jnp.int32),
            pltpu.VMEM((IPC, _L), jnp.float32),
            pltpu.VMEM((IPC + _L,), jnp.float32),
            pltpu.VMEM((C,), jnp.float32),
            pltpu.VMEM((_L,), jnp.float32),
            pltpu.SemaphoreType.DMA,
        ],
    )
    def fm(x_hbm, emb_hbm, lin_hbm, lb_hbm, out_hbm,
           x_v, idx_v, rows_v, lin_v, out_v, lb_v, sem):
        cid = lax.axis_index("c")
        sid = lax.axis_index("s")
        wid = sid * _NC + cid
        base = wid * BPW

        pltpu.sync_copy(lb_hbm, lb_v.at[pl.ds(0, 1)])
        lb = lb_v[pl.ds(0, _L)][0]

        lane = lax.iota(jnp.int32, _L)
        m_tail = jnp.where(lane < (F - _L), 1.0, 0.0).astype(jnp.float32)
        m_last = lane == (_L - 1)
        zeros_i = jnp.zeros((_L,), jnp.int32)

        def chunk_body(ci, carry):
            cbase = base + ci * C
            pltpu.sync_copy(x_hbm.at[pl.ds(cbase * F, IPC)], x_v)

            def idx_body(r, carry2):
                for j in range(128 // _L):
                    p0 = r * 128 + j * _L
                    pos = p0 + lane
                    fld = lax.rem(pos, F)
                    idx_v[r, pl.ds(j * _L, _L)] = x_v[pl.ds(p0, _L)] + fld * V
                return carry2
            lax.fori_loop(0, NSUB, idx_body, 0)

            copies = []
            for j in range(NSUB):
                copies.append(pltpu.async_copy(
                    emb_hbm.at[idx_v.at[j]],
                    rows_v.at[pl.ds(j * 128, 128)], sem))
                copies.append(pltpu.async_copy(
                    lin_hbm.at[idx_v.at[j]],
                    lin_v.at[pl.ds(j * 128, 128)], sem))
            for cp in copies:
                cp.wait()

            def row_body(b, carry2):
                rbase = b * F
                acc = rows_v[rbase, :]
                acc2 = acc * acc
                for f in range(1, F):
                    r = rows_v[rbase + f, :]
                    acc = acc + r
                    acc2 = acc2 + r * r
                lv = lin_v[pl.ds(rbase, _L)] + lin_v[pl.ds(rbase + _L, _L)] * m_tail
                cs = plsc.cumsum(acc)
                cq = plsc.cumsum(acc2)
                cl = plsc.cumsum(lv)
                logit = cl + lb + 0.5 * (cs * cs - cq)
                sig = 1.0 / (1.0 + jnp.exp(-logit))
                plsc.store_scatter(out_v, [zeros_i + b], sig, mask=m_last)
                return carry2
            lax.fori_loop(0, C, row_body, 0)

            pltpu.sync_copy(out_v, out_hbm.at[pl.ds(cbase, C)])
            return carry
        lax.fori_loop(0, NCH, chunk_body, 0)

    return fm


def _build_transpose(FV, D):
    NW = _NC * _NS
    W = 128
    J = 8
    CW = J * W
    NTC = FV // W
    TAIL = FV - NTC * W
    NB = NTC // J
    KMAX = (NB + NW - 1) // NW
    assert NTC % J == 0 and KMAX % 2 == 0 and TAIL * D % W == 0

    mesh = plsc.VectorSubcoreMesh(core_axis_name="c", subcore_axis_name="s")

    @functools.partial(
        pl.kernel,
        mesh=mesh,
        compiler_params=pltpu.CompilerParams(
            needs_layout_passes=False, use_tc_tiling_on_sc=True),
        out_type=jax.ShapeDtypeStruct((FV * D // W, W), jnp.float32),
        scratch_types=[
            pltpu.VMEM((2, D, CW), jnp.float32),
            pltpu.VMEM((2, CW * D // W, W), jnp.float32),
            pltpu.VMEM((TAIL * D // W, W), jnp.float32),
            pltpu.SemaphoreType.DMA,
            pltpu.SemaphoreType.DMA,
            pltpu.SemaphoreType.DMA,
            pltpu.SemaphoreType.DMA,
        ],
    )
    def transpose(src_hbm, tail_hbm, dst_hbm, tv, rv, lv, si0, si1, so0, so1):
        cid = lax.axis_index("c")
        sid = lax.axis_index("s")
        wid = sid * _NC + cid
        sin = (si0, si1)
        sout = (so0, so1)
        lane = lax.iota(jnp.int32, _L)
        RPB = CW * D // W

        def fire_in(m, p):
            @pl.when(m < NB)
            def _():
                pltpu.async_copy(
                    src_hbm.at[:, pl.ds(m * CW, CW)], tv.at[p], sin[p])

        def wait_in(m, p):
            @pl.when(m < NB)
            def _():
                pltpu.make_async_copy(
                    src_hbm.at[:, pl.ds(m * CW, CW)], tv.at[p], sin[p]).wait()

        def fire_out(m, p):
            @pl.when(m < NB)
            def _():
                pltpu.async_copy(
                    rv.at[p], dst_hbm.at[pl.ds(m * RPB, RPB)], sout[p])

        def wait_out(m, p):
            @pl.when(m < NB)
            def _():
                pltpu.make_async_copy(
                    rv.at[p], dst_hbm.at[pl.ds(m * RPB, RPB)],
                    sout[p]).wait()

        def trans(m, p):
            @pl.when(m < NB)
            def _():
                def step(i, carry):
                    for k in range(J):
                        r = i * J + k
                        vals = plsc.load_gather(tv.at[p], [lane, r + 0 * lane])
                        rv[p, i, pl.ds(k * D, D)] = vals
                    return carry
                lax.fori_loop(0, RPB, step, 0)

        def blk(k):
            return wid + NW * k

        fire_in(blk(0), 0)

        def pair_body(g, carry):
            k0 = 2 * g
            fire_in(blk(k0 + 1), 1)
            wait_in(blk(k0), 0)

            @pl.when(g > 0)
            def _():
                wait_out(blk(k0 - 2), 0)
            trans(blk(k0), 0)
            fire_out(blk(k0), 0)

            fire_in(blk(k0 + 2), 0)
            wait_in(blk(k0 + 1), 1)

            @pl.when(g > 0)
            def _():
                wait_out(blk(k0 - 1), 1)
            trans(blk(k0 + 1), 1)
            fire_out(blk(k0 + 1), 1)
            return carry
        lax.fori_loop(0, KMAX // 2, pair_body, 0)

        wait_in(blk(KMAX), 0)
        wait_out(blk(KMAX - 2), 0)
        wait_out(blk(KMAX - 1), 1)

        @pl.when(wid == NW - 1)
        def _():
            pltpu.sync_copy(tail_hbm, lv)
            pltpu.sync_copy(
                lv, dst_hbm.at[pl.ds(NTC * W * D // W, TAIL * D // W)])

    return transpose


def kernel(x, emb_table, lin_w, lin_b):
    B, F = x.shape
    D = emb_table.shape[1]
    V = emb_table.shape[0] // F
    FV = F * V
    TAIL = FV % 128
    tail = emb_table[FV - TAIL:, :].reshape(TAIL * D // 128, 128)
    w2 = _build_transpose(FV, D)(emb_table.T, tail)
    emb_2d = w2.reshape(FV, D)
    fm = _build(B, F, V, D)
    out = fm(x.reshape(B * F).astype(jnp.int32), emb_2d, lin_w, lin_b)
    return out.reshape(B, 1)


# speedup vs baseline: 1.2560x; 1.0008x over previous
import functools

import jax
import jax.numpy as jnp
from jax import lax
from jax.experimental import pallas as pl
from jax.experimental.pallas import tpu as pltpu
from jax.experimental.pallas import tpu_sc as plsc

_NC = 2
_NS = 16
_L = 16


def _build(B, F, V, D):
    NW = _NC * _NS
    BPW = B // NW
    C = 64
    NCH = BPW // C
    IPC = C * F
    NSUB = IPC // 128
    assert B % NW == 0 and BPW % C == 0 and IPC % 128 == 0

    mesh = plsc.VectorSubcoreMesh(core_axis_name="c", subcore_axis_name="s")

    @functools.partial(
        pl.kernel,
        mesh=mesh,
        compiler_params=pltpu.CompilerParams(
            needs_layout_passes=False, use_tc_tiling_on_sc=False),
        out_type=jax.ShapeDtypeStruct((B,), jnp.float32),
        scratch_types=[
            pltpu.VMEM((IPC,), jnp.int32),
            pltpu.VMEM((NSUB, 128), jnp.int32),
            pltpu.VMEM((IPC, _L), jnp.float32),
            pltpu.VMEM((IPC + _L,), jnp.float32),
            pltpu.VMEM((C,), jnp.float32),
            pltpu.VMEM((_L,), jnp.float32),
            pltpu.SemaphoreType.DMA,
        ],
    )
    def fm(x_hbm, emb_hbm, lin_hbm, lb_hbm, out_hbm,
           x_v, idx_v, rows_v, lin_v, out_v, lb_v, sem):
        cid = lax.axis_index("c")
        sid = lax.axis_index("s")
        wid = sid * _NC + cid
        base = wid * BPW

        pltpu.sync_copy(lb_hbm, lb_v.at[pl.ds(0, 1)])
        lb = lb_v[pl.ds(0, _L)][0]

        lane = lax.iota(jnp.int32, _L)
        m_tail = jnp.where(lane < (F - _L), 1.0, 0.0).astype(jnp.float32)
        m_last = lane == (_L - 1)
        zeros_i = jnp.zeros((_L,), jnp.int32)

        def chunk_body(ci, carry):
            cbase = base + ci * C
            pltpu.sync_copy(x_hbm.at[pl.ds(cbase * F, IPC)], x_v)

            def idx_body(r, carry2):
                for j in range(128 // _L):
                    p0 = r * 128 + j * _L
                    pos = p0 + lane
                    fld = lax.rem(pos, F)
                    idx_v[r, pl.ds(j * _L, _L)] = x_v[pl.ds(p0, _L)] + fld * V
                return carry2
            lax.fori_loop(0, NSUB, idx_body, 0)

            copies = []
            for j in range(NSUB):
                copies.append(pltpu.async_copy(
                    emb_hbm.at[idx_v.at[j]],
                    rows_v.at[pl.ds(j * 128, 128)], sem))
                copies.append(pltpu.async_copy(
                    lin_hbm.at[idx_v.at[j]],
                    lin_v.at[pl.ds(j * 128, 128)], sem))
            for cp in copies:
                cp.wait()

            def row_body(b, carry2):
                rbase = b * F
                acc = rows_v[rbase, :]
                acc2 = acc * acc
                for f in range(1, F):
                    r = rows_v[rbase + f, :]
                    acc = acc + r
                    acc2 = acc2 + r * r
                lv = lin_v[pl.ds(rbase, _L)] + lin_v[pl.ds(rbase + _L, _L)] * m_tail
                cs = plsc.cumsum(acc)
                cq = plsc.cumsum(acc2)
                cl = plsc.cumsum(lv)
                logit = cl + lb + 0.5 * (cs * cs - cq)
                sig = 1.0 / (1.0 + jnp.exp(-logit))
                plsc.store_scatter(out_v, [zeros_i + b], sig, mask=m_last)
                return carry2
            lax.fori_loop(0, C, row_body, 0)

            pltpu.sync_copy(out_v, out_hbm.at[pl.ds(cbase, C)])
            return carry
        lax.fori_loop(0, NCH, chunk_body, 0)

    return fm


def _build_transpose(FV, D):
    NW = _NC * _NS
    W = 128
    J = 8
    CW = J * W
    NTC = FV // W
    TAIL = FV - NTC * W
    NB = NTC // J
    KMAX = (NB + NW - 1) // NW
    assert NTC % J == 0 and KMAX % 2 == 0 and TAIL * D % W == 0

    mesh = plsc.VectorSubcoreMesh(core_axis_name="c", subcore_axis_name="s")

    @functools.partial(
        pl.kernel,
        mesh=mesh,
        compiler_params=pltpu.CompilerParams(
            needs_layout_passes=False, use_tc_tiling_on_sc=True),
        out_type=jax.ShapeDtypeStruct((FV * D // W, W), jnp.float32),
        scratch_types=[
            pltpu.VMEM((2, D, CW + 1), jnp.float32),
            pltpu.VMEM((2, CW * D // W, W), jnp.float32),
            pltpu.VMEM((TAIL * D // W, W), jnp.float32),
            pltpu.SemaphoreType.DMA,
            pltpu.SemaphoreType.DMA,
            pltpu.SemaphoreType.DMA,
            pltpu.SemaphoreType.DMA,
        ],
    )
    def transpose(src_hbm, tail_hbm, dst_hbm, tv, rv, lv, si0, si1, so0, so1):
        cid = lax.axis_index("c")
        sid = lax.axis_index("s")
        wid = sid * _NC + cid
        sin = (si0, si1)
        sout = (so0, so1)
        lane = lax.iota(jnp.int32, _L)
        RPB = CW * D // W

        def fire_in(m, p):
            @pl.when(m < NB)
            def _():
                pltpu.async_copy(
                    src_hbm.at[:, pl.ds(m * CW, CW)],
                    tv.at[p].at[:, pl.ds(0, CW)], sin[p])

        def wait_in(m, p):
            @pl.when(m < NB)
            def _():
                pltpu.make_async_copy(
                    src_hbm.at[:, pl.ds(m * CW, CW)],
                    tv.at[p].at[:, pl.ds(0, CW)], sin[p]).wait()

        def fire_out(m, p):
            @pl.when(m < NB)
            def _():
                pltpu.async_copy(
                    rv.at[p], dst_hbm.at[pl.ds(m * RPB, RPB)], sout[p])

        def wait_out(m, p):
            @pl.when(m < NB)
            def _():
                pltpu.make_async_copy(
                    rv.at[p], dst_hbm.at[pl.ds(m * RPB, RPB)],
                    sout[p]).wait()

        def trans(m, p):
            @pl.when(m < NB)
            def _():
                def step(i, carry):
                    for k in range(J):
                        r = i * J + k
                        vals = plsc.load_gather(tv.at[p], [lane, r + 0 * lane])
                        rv[p, i, pl.ds(k * D, D)] = vals
                    return carry
                lax.fori_loop(0, RPB, step, 0)

        def blk(k):
            return wid + NW * k

        fire_in(blk(0), 0)

        def pair_body(g, carry):
            k0 = 2 * g
            fire_in(blk(k0 + 1), 1)
            wait_in(blk(k0), 0)

            @pl.when(g > 0)
            def _():
                wait_out(blk(k0 - 2), 0)
            trans(blk(k0), 0)
            fire_out(blk(k0), 0)

            fire_in(blk(k0 + 2), 0)
            wait_in(blk(k0 + 1), 1)

            @pl.when(g > 0)
            def _():
                wait_out(blk(k0 - 1), 1)
            trans(blk(k0 + 1), 1)
            fire_out(blk(k0 + 1), 1)
            return carry
        lax.fori_loop(0, KMAX // 2, pair_body, 0)

        wait_in(blk(KMAX), 0)
        wait_out(blk(KMAX - 2), 0)
        wait_out(blk(KMAX - 1), 1)

        @pl.when(wid == NW - 1)
        def _():
            pltpu.sync_copy(tail_hbm, lv)
            pltpu.sync_copy(
                lv, dst_hbm.at[pl.ds(NTC * W * D // W, TAIL * D // W)])

    return transpose


def kernel(x, emb_table, lin_w, lin_b):
    B, F = x.shape
    D = emb_table.shape[1]
    V = emb_table.shape[0] // F
    FV = F * V
    TAIL = FV % 128
    tail = emb_table[FV - TAIL:, :].reshape(TAIL * D // 128, 128)
    w2 = _build_transpose(FV, D)(emb_table.T, tail)
    emb_2d = w2.reshape(FV, D)
    fm = _build(B, F, V, D)
    out = fm(x.reshape(B * F).astype(jnp.int32), emb_2d, lin_w, lin_b)
    return out.reshape(B, 1)
